# baseline (speedup 1.0000x reference)
import functools

import jax
import jax.numpy as jnp
from jax import lax
from jax.experimental import pallas as pl
from jax.experimental.pallas import tpu as pltpu
from jax.experimental.pallas import tpu_sc as plsc

N = 10000
E = 320000
D = 128
NPAD = 10240
STRIP = NPAD // 16
NC, NS = 2, 16
NW = NC * NS
B = 128
EPAD = NW * 80 * B
EPT = EPAD // NW
RPT = EPT // B
KCH = 1
CE = EPAD // KCH
CRPT = RPT // KCH

_MESH = plsc.VectorSubcoreMesh(core_axis_name="c", subcore_axis_name="s")
_f32 = jnp.float32
_bf16 = jnp.bfloat16


def _wid():
    return lax.axis_index("s") * NC + lax.axis_index("c")


@functools.partial(
    pl.kernel,
    out_type=(jax.ShapeDtypeStruct((NPAD,), _f32),
              jax.ShapeDtypeStruct((NPAD,), _f32)),
    mesh=_MESH,
    scratch_types=[
        pltpu.VMEM((RPT, B), jnp.int32),
        pltpu.VMEM((B,), _f32),
        pltpu.VMEM_SHARED((NPAD,), _f32),
    ],
)
def _sc_degree(dst3d, zeros1, out0, out1, didx, ones_v, acc):
    c = lax.axis_index("c")
    s = lax.axis_index("s")
    pltpu.sync_copy(zeros1, acc.at[pl.ds(s * STRIP, STRIP)])
    pltpu.sync_copy(dst3d.at[_wid()], didx)
    for k in range(B // 16):
        ones_v[pl.ds(k * 16, 16)] = jnp.ones((16,), _f32)
    plsc.subcore_barrier()

    def body(j, carry):
        pltpu.sync_copy(ones_v, acc.at[didx.at[j]], add=True)
        return carry

    lax.fori_loop(0, RPT, body, 0)
    plsc.subcore_barrier()

    @pl.when(c == 0)
    def _():
        pltpu.sync_copy(acc.at[pl.ds(s * STRIP, STRIP)],
                        out0.at[pl.ds(s * STRIP, STRIP)])

    @pl.when(c == 1)
    def _():
        pltpu.sync_copy(acc.at[pl.ds(s * STRIP, STRIP)],
                        out1.at[pl.ds(s * STRIP, STRIP)])


@functools.partial(
    pl.kernel,
    out_type=jax.ShapeDtypeStruct((NC, NPAD, D), _f32),
    mesh=_MESH,
    scratch_types=[
        pltpu.VMEM((RPT, B), jnp.int32),
        pltpu.VMEM((RPT, B), jnp.int32),
        pltpu.VMEM((B, D), _f32),
        pltpu.VMEM_SHARED((NPAD, D), _f32),
        pltpu.SemaphoreType.DMA,
    ],
)
def _sc_agg(y, src3d, dst3d, out, sidx, didx, rows, acc, sem):
    c = lax.axis_index("c")
    s = lax.axis_index("s")
    w = _wid()
    for i in range(B):
        for k in range(D // 16):
            rows[i, pl.ds(k * 16, 16)] = jnp.zeros((16,), _f32)
    for k in range(STRIP // B):
        pltpu.sync_copy(rows, acc.at[pl.ds(s * STRIP + k * B, B)])
    pltpu.sync_copy(src3d.at[w], sidx)
    pltpu.sync_copy(dst3d.at[w], didx)
    plsc.subcore_barrier()

    def body(j, carry):
        pltpu.async_copy(y.at[sidx.at[j]], rows, sem).wait()
        pltpu.sync_copy(rows, acc.at[didx.at[j]], add=True)
        return carry

    lax.fori_loop(0, RPT, body, 0)
    plsc.subcore_barrier()
    pltpu.sync_copy(acc.at[pl.ds(s * STRIP, STRIP)],
                    out.at[c, pl.ds(s * STRIP, STRIP)])


@functools.partial(
    pl.kernel,
    out_type=(jax.ShapeDtypeStruct((CE, D), _f32),
              jax.ShapeDtypeStruct((CE, D), _f32)),
    mesh=_MESH,
    scratch_types=(
        [pltpu.VMEM((CRPT, B), jnp.int32)] * 2
        + [pltpu.VMEM((B, D), _f32)] * 6
        + [pltpu.SemaphoreType.DMA] * 12
    ),
)
def _sc_gather2(p, q, src3d, dst3d, pg, qg,
                sidx, didx, p0, p1, p2, q0, q1, q2,
                gp0, gp1, gp2, gq0, gq1, gq2,
                wp0, wp1, wp2, wq0, wq1, wq2):
    w = _wid()
    pltpu.sync_copy(src3d.at[w], sidx)
    pltpu.sync_copy(dst3d.at[w], didx)
    base = w * (CE // NW)
    pb = (p0, p1, p2)
    qb = (q0, q1, q2)
    gp = (gp0, gp1, gp2)
    gq = (gq0, gq1, gq2)
    wp = (wp0, wp1, wp2)
    wq = (wq0, wq1, wq2)

    for t in range(3):
        pltpu.async_copy(p.at[sidx.at[t]], pb[t], gp[t])
        pltpu.async_copy(q.at[didx.at[t]], qb[t], gq[t])

    def body(g, carry):
        j = 3 * g
        for t in range(3):
            pltpu.make_async_copy(p.at[sidx.at[j + t]], pb[t], gp[t]).wait()
            pltpu.async_copy(pb[t], pg.at[pl.ds(base + (j + t) * B, B)], wp[t])
            pltpu.make_async_copy(q.at[didx.at[j + t]], qb[t], gq[t]).wait()
            pltpu.async_copy(qb[t], qg.at[pl.ds(base + (j + t) * B, B)], wq[t])
        for t in range(3):
            pltpu.make_async_copy(pb[t], pg.at[pl.ds(base, B)], wp[t]).wait()
            pltpu.async_copy(p.at[sidx.at[j + 3 + t]], pb[t], gp[t])
            pltpu.make_async_copy(qb[t], qg.at[pl.ds(base, B)], wq[t]).wait()
            pltpu.async_copy(q.at[didx.at[j + 3 + t]], qb[t], gq[t])
        return carry

    lax.fori_loop(0, CRPT // 3 - 1, body, 0)
    j = CRPT - 5
    for t in range(3):
        pltpu.make_async_copy(p.at[sidx.at[j + t]], pb[t], gp[t]).wait()
        pltpu.async_copy(pb[t], pg.at[pl.ds(base + (j + t) * B, B)], wp[t])
        pltpu.make_async_copy(q.at[didx.at[j + t]], qb[t], gq[t]).wait()
        pltpu.async_copy(qb[t], qg.at[pl.ds(base + (j + t) * B, B)], wq[t])
    for t in range(2):
        pltpu.make_async_copy(pb[t], pg.at[pl.ds(base, B)], wp[t]).wait()
        pltpu.async_copy(p.at[sidx.at[j + 3 + t]], pb[t], gp[t])
        pltpu.make_async_copy(qb[t], qg.at[pl.ds(base, B)], wq[t]).wait()
        pltpu.async_copy(q.at[didx.at[j + 3 + t]], qb[t], gq[t])
    for t in range(2):
        pltpu.make_async_copy(p.at[sidx.at[j + 3 + t]], pb[t], gp[t]).wait()
        pltpu.async_copy(pb[t], pg.at[pl.ds(base + (j + 3 + t) * B, B)], wp[t])
        pltpu.make_async_copy(q.at[didx.at[j + 3 + t]], qb[t], gq[t]).wait()
        pltpu.async_copy(qb[t], qg.at[pl.ds(base + (j + 3 + t) * B, B)], wq[t])
    for t in range(3):
        pltpu.make_async_copy(pb[t], pg.at[pl.ds(base, B)], wp[t]).wait()
        pltpu.make_async_copy(qb[t], qg.at[pl.ds(base, B)], wq[t]).wait()


_R = 640


def _prep_body(x_ref, w_ref, d0_ref, d1_ref, y_ref, dis_ref):
    dis = lax.rsqrt(d0_ref[...] + d1_ref[...] + 1.0)
    y_ref[...] = jnp.dot(x_ref[...], w_ref[...],
                         preferred_element_type=_f32) * dis
    dis_ref[...] = dis


def _tc_prep(x_pad, W1, d0, d1):
    return pl.pallas_call(
        _prep_body,
        grid=(NPAD // _R,),
        in_specs=[
            pl.BlockSpec((_R, D), lambda i: (i, 0)),
            pl.BlockSpec((D, D), lambda i: (0, 0)),
            pl.BlockSpec((_R, 1), lambda i: (i, 0)),
            pl.BlockSpec((_R, 1), lambda i: (i, 0)),
        ],
        out_specs=[
            pl.BlockSpec((_R, D), lambda i: (i, 0)),
            pl.BlockSpec((_R, 1), lambda i: (i, 0)),
        ],
        out_shape=[
            jax.ShapeDtypeStruct((NPAD, D), _f32),
            jax.ShapeDtypeStruct((NPAD, 1), _f32),
        ],
    )(x_pad, W1, d0, d1)


def _layer_body(a0_ref, a1_ref, y_ref, dis_ref, b_ref, w_ref, o_ref):
    h = jnp.maximum(
        dis_ref[...] * (a0_ref[0] + a1_ref[0] + y_ref[...]) + b_ref[...], 0.0)
    o_ref[...] = jnp.dot(h, w_ref[...], preferred_element_type=_f32) * dis_ref[...]


def _tc_layer(agg, y, dis, b, W):
    return pl.pallas_call(
        _layer_body,
        grid=(NPAD // _R,),
        in_specs=[
            pl.BlockSpec((1, _R, D), lambda i: (0, i, 0)),
            pl.BlockSpec((1, _R, D), lambda i: (1, i, 0)),
            pl.BlockSpec((_R, D), lambda i: (i, 0)),
            pl.BlockSpec((_R, 1), lambda i: (i, 0)),
            pl.BlockSpec((1, D), lambda i: (0, 0)),
            pl.BlockSpec((D, D), lambda i: (0, 0)),
        ],
        out_specs=pl.BlockSpec((_R, D), lambda i: (i, 0)),
        out_shape=jax.ShapeDtypeStruct((NPAD, D), _f32),
    )(agg, agg, y, dis, b, W)


def _pq_body(a0_ref, a1_ref, y_ref, dis_ref, b_ref, wa_ref, wb_ref,
             p_ref, q_ref):
    h = jnp.maximum(
        dis_ref[...] * (a0_ref[0] + a1_ref[0] + y_ref[...]) + b_ref[...], 0.0)
    p_ref[...] = jnp.dot(h, wa_ref[...], preferred_element_type=_f32)
    q_ref[...] = jnp.dot(h, wb_ref[...], preferred_element_type=_f32)


def _tc_pq(agg, y, dis, b, WA, WB):
    return pl.pallas_call(
        _pq_body,
        grid=(NPAD // _R,),
        in_specs=[
            pl.BlockSpec((1, _R, D), lambda i: (0, i, 0)),
            pl.BlockSpec((1, _R, D), lambda i: (1, i, 0)),
            pl.BlockSpec((_R, D), lambda i: (i, 0)),
            pl.BlockSpec((_R, 1), lambda i: (i, 0)),
            pl.BlockSpec((1, D), lambda i: (0, 0)),
            pl.BlockSpec((D, D), lambda i: (0, 0)),
            pl.BlockSpec((D, D), lambda i: (0, 0)),
        ],
        out_specs=[
            pl.BlockSpec((_R, D), lambda i: (i, 0)),
            pl.BlockSpec((_R, D), lambda i: (i, 0)),
        ],
        out_shape=[
            jax.ShapeDtypeStruct((NPAD, D), _f32),
            jax.ShapeDtypeStruct((NPAD, D), _f32),
        ],
    )(agg, agg, y, dis, b, WA, WB)


_EB = 2000


def _edge_body(pg_ref, qg_ref, ef_ref, wc_ref, b1_ref, w2_ref, b2_ref,
               w3_ref, b3_ref, o_ref):
    z = pg_ref[...] + qg_ref[...] + jnp.dot(
        ef_ref[...], wc_ref[...], preferred_element_type=_f32) + b1_ref[...]
    z = jnp.maximum(z, 0.0)
    z = jnp.maximum(
        jnp.dot(z, w2_ref[...], preferred_element_type=_f32) + b2_ref[...], 0.0)
    o_ref[...] = jnp.dot(z, w3_ref[...], preferred_element_type=_f32) + b3_ref[...]


def _tc_edge(pg, qg, ef, WC, bm1, Wm2, bm2, Wm3, bm3):
    return pl.pallas_call(
        _edge_body,
        grid=(E // _EB,),
        in_specs=[
            pl.BlockSpec((_EB, D), lambda i: (i, 0)),
            pl.BlockSpec((_EB, D), lambda i: (i, 0)),
            pl.BlockSpec((_EB, 16), lambda i: (i, 0)),
            pl.BlockSpec((16, D), lambda i: (0, 0)),
            pl.BlockSpec((1, D), lambda i: (0, 0)),
            pl.BlockSpec((D, 64), lambda i: (0, 0)),
            pl.BlockSpec((1, 64), lambda i: (0, 0)),
            pl.BlockSpec((64, 1), lambda i: (0, 0)),
            pl.BlockSpec((1, 1), lambda i: (0, 0)),
        ],
        out_specs=pl.BlockSpec((_EB, 1), lambda i: (i, 0)),
        out_shape=jax.ShapeDtypeStruct((E, 1), _f32),
    )(pg, qg, ef, WC, bm1, Wm2, bm2, Wm3, bm3)


def kernel(x, edge_index, edge_feat, W1, b1, W2, b2, Wm1, bm1, Wm2, bm2,
           Wm3, bm3):
    npad_e = EPAD - E
    pad_iota = jnp.arange(npad_e, dtype=jnp.int32)
    src_pad = jnp.concatenate([edge_index[0], pad_iota % N])
    dst_pad = jnp.concatenate([edge_index[1], N + pad_iota % (NPAD - N)])
    src3d = src_pad.reshape(NW, RPT, B)
    dst3d = dst_pad.reshape(NW, RPT, B)
    x_pad = jnp.pad(x, ((0, NPAD - N), (0, 0)))
    zeros1 = jnp.zeros((STRIP,), _f32)

    d0, d1 = _sc_degree(dst3d, zeros1)
    y1, dis = _tc_prep(x_pad, W1, d0.reshape(NPAD, 1), d1.reshape(NPAD, 1))
    agg1 = _sc_agg(y1, src3d, dst3d)
    y2 = _tc_layer(agg1, y1, dis, b1.reshape(1, D), W2)
    agg2 = _sc_agg(y2, src3d, dst3d)
    p, q = _tc_pq(agg2, y2, dis, b2.reshape(1, D), Wm1[:D], Wm1[D:2 * D])
    pg, qg = _sc_gather2(p, q, src3d, dst3d)
    return _tc_edge(pg, qg, edge_feat, Wm1[2 * D:], bm1.reshape(1, D),
                    Wm2, bm2.reshape(1, 64), Wm3, bm3.reshape(1, 1))


# speedup vs baseline: 1.0437x; 1.0437x over previous
import functools

import jax
import jax.numpy as jnp
from jax import lax
from jax.experimental import pallas as pl
from jax.experimental.pallas import tpu as pltpu
from jax.experimental.pallas import tpu_sc as plsc

N = 10000
E = 320000
D = 128
NPAD = 10240
STRIP = NPAD // 16
NC, NS = 2, 16
NW = NC * NS
B = 128
EPAD = NW * 80 * B
EPT = EPAD // NW
RPT = EPT // B
KCH = 1
CE = EPAD // KCH
CRPT = RPT // KCH

_MESH = plsc.VectorSubcoreMesh(core_axis_name="c", subcore_axis_name="s")
_f32 = jnp.float32
_bf16 = jnp.bfloat16


def _wid():
    return lax.axis_index("s") * NC + lax.axis_index("c")


@functools.partial(
    pl.kernel,
    out_type=(jax.ShapeDtypeStruct((NPAD,), _f32),
              jax.ShapeDtypeStruct((NPAD,), _f32)),
    mesh=_MESH,
    scratch_types=[
        pltpu.VMEM((RPT, B), jnp.int32),
        pltpu.VMEM((B,), _f32),
        pltpu.VMEM_SHARED((NPAD,), _f32),
    ],
)
def _sc_degree(dst3d, zeros1, out0, out1, didx, ones_v, acc):
    c = lax.axis_index("c")
    s = lax.axis_index("s")
    pltpu.sync_copy(zeros1, acc.at[pl.ds(s * STRIP, STRIP)])
    pltpu.sync_copy(dst3d.at[_wid()], didx)
    for k in range(B // 16):
        ones_v[pl.ds(k * 16, 16)] = jnp.ones((16,), _f32)
    plsc.subcore_barrier()

    def body(j, carry):
        pltpu.sync_copy(ones_v, acc.at[didx.at[j]], add=True)
        return carry

    lax.fori_loop(0, RPT, body, 0)
    plsc.subcore_barrier()

    @pl.when(c == 0)
    def _():
        pltpu.sync_copy(acc.at[pl.ds(s * STRIP, STRIP)],
                        out0.at[pl.ds(s * STRIP, STRIP)])

    @pl.when(c == 1)
    def _():
        pltpu.sync_copy(acc.at[pl.ds(s * STRIP, STRIP)],
                        out1.at[pl.ds(s * STRIP, STRIP)])


@functools.partial(
    pl.kernel,
    out_type=jax.ShapeDtypeStruct((NC, NPAD, D), _f32),
    mesh=_MESH,
    scratch_types=[
        pltpu.VMEM((RPT, B), jnp.int32),
        pltpu.VMEM((RPT, B), jnp.int32),
        pltpu.VMEM((B, D), _f32),
        pltpu.VMEM_SHARED((NPAD, D), _f32),
        pltpu.SemaphoreType.DMA,
    ],
)
def _sc_agg(y, src3d, dst3d, out, sidx, didx, rows, acc, sem):
    c = lax.axis_index("c")
    s = lax.axis_index("s")
    w = _wid()
    for i in range(B):
        for k in range(D // 16):
            rows[i, pl.ds(k * 16, 16)] = jnp.zeros((16,), _f32)
    for k in range(STRIP // B):
        pltpu.sync_copy(rows, acc.at[pl.ds(s * STRIP + k * B, B)])
    pltpu.sync_copy(src3d.at[w], sidx)
    pltpu.sync_copy(dst3d.at[w], didx)
    plsc.subcore_barrier()

    def body(j, carry):
        pltpu.async_copy(y.at[sidx.at[j]], rows, sem).wait()
        pltpu.sync_copy(rows, acc.at[didx.at[j]], add=True)
        return carry

    lax.fori_loop(0, RPT, body, 0)
    plsc.subcore_barrier()
    pltpu.sync_copy(acc.at[pl.ds(s * STRIP, STRIP)],
                    out.at[c, pl.ds(s * STRIP, STRIP)])


@functools.partial(
    pl.kernel,
    out_type=(jax.ShapeDtypeStruct((CE, D), _f32),
              jax.ShapeDtypeStruct((CE, D), _f32)),
    mesh=_MESH,
    scratch_types=[
        pltpu.VMEM((CRPT, B), jnp.int32),
        pltpu.VMEM((CRPT, B), jnp.int32),
        pltpu.VMEM((B, D), _f32),
        pltpu.VMEM((B, D), _f32),
        pltpu.VMEM((B, D), _f32),
        pltpu.VMEM((B, D), _f32),
        pltpu.SemaphoreType.DMA,
        pltpu.SemaphoreType.DMA,
        pltpu.SemaphoreType.DMA,
        pltpu.SemaphoreType.DMA,
    ],
)
def _sc_gather2(p, q, src3d, dst3d, pg, qg,
                sidx, didx, bp0, bp1, bq0, bq1, sp0, sp1, sq0, sq1):
    w = _wid()
    pltpu.sync_copy(src3d.at[w], sidx)
    pltpu.sync_copy(dst3d.at[w], didx)
    base = w * (CE // NW)

    pltpu.async_copy(p.at[sidx.at[0]], bp0, sp0)
    pltpu.async_copy(q.at[didx.at[0]], bq0, sq0)
    pltpu.async_copy(p.at[sidx.at[1]], bp1, sp1)
    pltpu.async_copy(q.at[didx.at[1]], bq1, sq1)

    def body(j2, carry):
        j = 2 * j2
        pltpu.make_async_copy(p.at[sidx.at[j]], bp0, sp0).wait()
        pltpu.sync_copy(bp0, pg.at[pl.ds(base + j * B, B)])
        pltpu.async_copy(p.at[sidx.at[j + 2]], bp0, sp0)
        pltpu.make_async_copy(q.at[didx.at[j]], bq0, sq0).wait()
        pltpu.sync_copy(bq0, qg.at[pl.ds(base + j * B, B)])
        pltpu.async_copy(q.at[didx.at[j + 2]], bq0, sq0)
        pltpu.make_async_copy(p.at[sidx.at[j + 1]], bp1, sp1).wait()
        pltpu.sync_copy(bp1, pg.at[pl.ds(base + (j + 1) * B, B)])
        pltpu.async_copy(p.at[sidx.at[j + 3]], bp1, sp1)
        pltpu.make_async_copy(q.at[didx.at[j + 1]], bq1, sq1).wait()
        pltpu.sync_copy(bq1, qg.at[pl.ds(base + (j + 1) * B, B)])
        pltpu.async_copy(q.at[didx.at[j + 3]], bq1, sq1)
        return carry

    lax.fori_loop(0, CRPT // 2 - 1, body, 0)
    j = CRPT - 2
    pltpu.make_async_copy(p.at[sidx.at[j]], bp0, sp0).wait()
    pltpu.sync_copy(bp0, pg.at[pl.ds(base + j * B, B)])
    pltpu.make_async_copy(q.at[didx.at[j]], bq0, sq0).wait()
    pltpu.sync_copy(bq0, qg.at[pl.ds(base + j * B, B)])
    pltpu.make_async_copy(p.at[sidx.at[j + 1]], bp1, sp1).wait()
    pltpu.sync_copy(bp1, pg.at[pl.ds(base + (j + 1) * B, B)])
    pltpu.make_async_copy(q.at[didx.at[j + 1]], bq1, sq1).wait()
    pltpu.sync_copy(bq1, qg.at[pl.ds(base + (j + 1) * B, B)])


_R = 640


def _prep_body(x_ref, w_ref, d0_ref, d1_ref, y_ref, dis_ref):
    dis = lax.rsqrt(d0_ref[...] + d1_ref[...] + 1.0)
    y_ref[...] = jnp.dot(x_ref[...], w_ref[...],
                         preferred_element_type=_f32) * dis
    dis_ref[...] = dis


def _tc_prep(x_pad, W1, d0, d1):
    return pl.pallas_call(
        _prep_body,
        grid=(NPAD // _R,),
        in_specs=[
            pl.BlockSpec((_R, D), lambda i: (i, 0)),
            pl.BlockSpec((D, D), lambda i: (0, 0)),
            pl.BlockSpec((_R, 1), lambda i: (i, 0)),
            pl.BlockSpec((_R, 1), lambda i: (i, 0)),
        ],
        out_specs=[
            pl.BlockSpec((_R, D), lambda i: (i, 0)),
            pl.BlockSpec((_R, 1), lambda i: (i, 0)),
        ],
        out_shape=[
            jax.ShapeDtypeStruct((NPAD, D), _f32),
            jax.ShapeDtypeStruct((NPAD, 1), _f32),
        ],
    )(x_pad, W1, d0, d1)


def _layer_body(a0_ref, a1_ref, y_ref, dis_ref, b_ref, w_ref, o_ref):
    h = jnp.maximum(
        dis_ref[...] * (a0_ref[0] + a1_ref[0] + y_ref[...]) + b_ref[...], 0.0)
    o_ref[...] = jnp.dot(h, w_ref[...], preferred_element_type=_f32) * dis_ref[...]


def _tc_layer(agg, y, dis, b, W):
    return pl.pallas_call(
        _layer_body,
        grid=(NPAD // _R,),
        in_specs=[
            pl.BlockSpec((1, _R, D), lambda i: (0, i, 0)),
            pl.BlockSpec((1, _R, D), lambda i: (1, i, 0)),
            pl.BlockSpec((_R, D), lambda i: (i, 0)),
            pl.BlockSpec((_R, 1), lambda i: (i, 0)),
            pl.BlockSpec((1, D), lambda i: (0, 0)),
            pl.BlockSpec((D, D), lambda i: (0, 0)),
        ],
        out_specs=pl.BlockSpec((_R, D), lambda i: (i, 0)),
        out_shape=jax.ShapeDtypeStruct((NPAD, D), _f32),
    )(agg, agg, y, dis, b, W)


def _pq_body(a0_ref, a1_ref, y_ref, dis_ref, b_ref, wa_ref, wb_ref,
             p_ref, q_ref):
    h = jnp.maximum(
        dis_ref[...] * (a0_ref[0] + a1_ref[0] + y_ref[...]) + b_ref[...], 0.0)
    p_ref[...] = jnp.dot(h, wa_ref[...], preferred_element_type=_f32)
    q_ref[...] = jnp.dot(h, wb_ref[...], preferred_element_type=_f32)


def _tc_pq(agg, y, dis, b, WA, WB):
    return pl.pallas_call(
        _pq_body,
        grid=(NPAD // _R,),
        in_specs=[
            pl.BlockSpec((1, _R, D), lambda i: (0, i, 0)),
            pl.BlockSpec((1, _R, D), lambda i: (1, i, 0)),
            pl.BlockSpec((_R, D), lambda i: (i, 0)),
            pl.BlockSpec((_R, 1), lambda i: (i, 0)),
            pl.BlockSpec((1, D), lambda i: (0, 0)),
            pl.BlockSpec((D, D), lambda i: (0, 0)),
            pl.BlockSpec((D, D), lambda i: (0, 0)),
        ],
        out_specs=[
            pl.BlockSpec((_R, D), lambda i: (i, 0)),
            pl.BlockSpec((_R, D), lambda i: (i, 0)),
        ],
        out_shape=[
            jax.ShapeDtypeStruct((NPAD, D), _f32),
            jax.ShapeDtypeStruct((NPAD, D), _f32),
        ],
    )(agg, agg, y, dis, b, WA, WB)


_EB = 4000


def _edge_body(pg_ref, qg_ref, ef_ref, wc_ref, b1_ref, w2_ref, b2_ref,
               w3_ref, b3_ref, o_ref):
    z = pg_ref[...] + qg_ref[...] + jnp.dot(
        ef_ref[...], wc_ref[...], preferred_element_type=_f32) + b1_ref[...]
    z = jnp.maximum(z, 0.0)
    z = jnp.maximum(
        jnp.dot(z, w2_ref[...], preferred_element_type=_f32) + b2_ref[...], 0.0)
    o_ref[...] = jnp.dot(z, w3_ref[...], preferred_element_type=_f32) + b3_ref[...]


def _tc_edge(pg, qg, ef, WC, bm1, Wm2, bm2, Wm3, bm3):
    return pl.pallas_call(
        _edge_body,
        grid=(E // _EB,),
        in_specs=[
            pl.BlockSpec((_EB, D), lambda i: (i, 0)),
            pl.BlockSpec((_EB, D), lambda i: (i, 0)),
            pl.BlockSpec((_EB, 16), lambda i: (i, 0)),
            pl.BlockSpec((16, D), lambda i: (0, 0)),
            pl.BlockSpec((1, D), lambda i: (0, 0)),
            pl.BlockSpec((D, 64), lambda i: (0, 0)),
            pl.BlockSpec((1, 64), lambda i: (0, 0)),
            pl.BlockSpec((64, 1), lambda i: (0, 0)),
            pl.BlockSpec((1, 1), lambda i: (0, 0)),
        ],
        out_specs=pl.BlockSpec((_EB, 1), lambda i: (i, 0)),
        out_shape=jax.ShapeDtypeStruct((E, 1), _f32),
    )(pg, qg, ef, WC, bm1, Wm2, bm2, Wm3, bm3)


def kernel(x, edge_index, edge_feat, W1, b1, W2, b2, Wm1, bm1, Wm2, bm2,
           Wm3, bm3):
    npad_e = EPAD - E
    pad_iota = jnp.arange(npad_e, dtype=jnp.int32)
    src_pad = jnp.concatenate([edge_index[0], pad_iota % N])
    dst_pad = jnp.concatenate([edge_index[1], N + pad_iota % (NPAD - N)])
    src3d = src_pad.reshape(NW, RPT, B)
    dst3d = dst_pad.reshape(NW, RPT, B)
    x_pad = jnp.pad(x, ((0, NPAD - N), (0, 0)))
    zeros1 = jnp.zeros((STRIP,), _f32)

    d0, d1 = _sc_degree(dst3d, zeros1)
    y1, dis = _tc_prep(x_pad, W1, d0.reshape(NPAD, 1), d1.reshape(NPAD, 1))
    agg1 = _sc_agg(y1, src3d, dst3d)
    y2 = _tc_layer(agg1, y1, dis, b1.reshape(1, D), W2)
    agg2 = _sc_agg(y2, src3d, dst3d)
    p, q = _tc_pq(agg2, y2, dis, b2.reshape(1, D), Wm1[:D], Wm1[D:2 * D])
    pg, qg = _sc_gather2(p, q, src3d, dst3d)
    return _tc_edge(pg, qg, edge_feat, Wm1[2 * D:], bm1.reshape(1, D),
                    Wm2, bm2.reshape(1, 64), Wm3, bm3.reshape(1, 1))


# speedup vs baseline: 1.0616x; 1.0172x over previous
import functools

import jax
import jax.numpy as jnp
from jax import lax
from jax.experimental import pallas as pl
from jax.experimental.pallas import tpu as pltpu
from jax.experimental.pallas import tpu_sc as plsc

N = 10000
E = 320000
D = 128
NPAD = 10240
STRIP = NPAD // 16
NC, NS = 2, 16
NW = NC * NS
B = 128
EPAD = NW * 80 * B
EPT = EPAD // NW
RPT = EPT // B
KCH = 1
CE = EPAD // KCH
CRPT = RPT // KCH

_MESH = plsc.VectorSubcoreMesh(core_axis_name="c", subcore_axis_name="s")
_f32 = jnp.float32
_bf16 = jnp.bfloat16


def _wid():
    return lax.axis_index("s") * NC + lax.axis_index("c")


@functools.partial(
    pl.kernel,
    out_type=(jax.ShapeDtypeStruct((NPAD,), _f32),
              jax.ShapeDtypeStruct((NPAD,), _f32)),
    mesh=_MESH,
    scratch_types=[
        pltpu.VMEM((RPT, B), jnp.int32),
        pltpu.VMEM((B,), _f32),
        pltpu.VMEM_SHARED((NPAD,), _f32),
    ],
)
def _sc_degree(dst3d, zeros1, out0, out1, didx, ones_v, acc):
    c = lax.axis_index("c")
    s = lax.axis_index("s")
    pltpu.sync_copy(zeros1, acc.at[pl.ds(s * STRIP, STRIP)])
    pltpu.sync_copy(dst3d.at[_wid()], didx)
    for k in range(B // 16):
        ones_v[pl.ds(k * 16, 16)] = jnp.ones((16,), _f32)
    plsc.subcore_barrier()

    def body(j, carry):
        pltpu.sync_copy(ones_v, acc.at[didx.at[j]], add=True)
        return carry

    lax.fori_loop(0, RPT, body, 0)
    plsc.subcore_barrier()

    @pl.when(c == 0)
    def _():
        pltpu.sync_copy(acc.at[pl.ds(s * STRIP, STRIP)],
                        out0.at[pl.ds(s * STRIP, STRIP)])

    @pl.when(c == 1)
    def _():
        pltpu.sync_copy(acc.at[pl.ds(s * STRIP, STRIP)],
                        out1.at[pl.ds(s * STRIP, STRIP)])


@functools.partial(
    pl.kernel,
    out_type=jax.ShapeDtypeStruct((NC, NPAD, D), _f32),
    mesh=_MESH,
    scratch_types=[
        pltpu.VMEM((RPT, B), jnp.int32),
        pltpu.VMEM((RPT, B), jnp.int32),
        pltpu.VMEM((B, D), _f32),
        pltpu.VMEM_SHARED((NPAD, D), _f32),
        pltpu.SemaphoreType.DMA,
    ],
)
def _sc_agg(y, src3d, dst3d, out, sidx, didx, rows, acc, sem):
    c = lax.axis_index("c")
    s = lax.axis_index("s")
    w = _wid()
    for i in range(B):
        for k in range(D // 16):
            rows[i, pl.ds(k * 16, 16)] = jnp.zeros((16,), _f32)
    for k in range(STRIP // B):
        pltpu.sync_copy(rows, acc.at[pl.ds(s * STRIP + k * B, B)])
    pltpu.sync_copy(src3d.at[w], sidx)
    pltpu.sync_copy(dst3d.at[w], didx)
    plsc.subcore_barrier()

    def body(j, carry):
        pltpu.async_copy(y.at[sidx.at[j]], rows, sem).wait()
        pltpu.sync_copy(rows, acc.at[didx.at[j]], add=True)
        return carry

    lax.fori_loop(0, RPT, body, 0)
    plsc.subcore_barrier()
    pltpu.sync_copy(acc.at[pl.ds(s * STRIP, STRIP)],
                    out.at[c, pl.ds(s * STRIP, STRIP)])


@functools.partial(
    pl.kernel,
    out_type=(jax.ShapeDtypeStruct((CE, D), _f32),
              jax.ShapeDtypeStruct((CE, D), _f32)),
    mesh=_MESH,
    scratch_types=[
        pltpu.VMEM((CRPT, B), jnp.int32),
        pltpu.VMEM((CRPT, B), jnp.int32),
        pltpu.VMEM((B, D), _f32),
        pltpu.VMEM((B, D), _f32),
        pltpu.VMEM((B, D), _f32),
        pltpu.VMEM((B, D), _f32),
        pltpu.SemaphoreType.DMA,
        pltpu.SemaphoreType.DMA,
        pltpu.SemaphoreType.DMA,
        pltpu.SemaphoreType.DMA,
    ],
)
def _sc_gather2(p, q, src3d, dst3d, pg, qg,
                sidx, didx, bp0, bp1, bq0, bq1, sp0, sp1, sq0, sq1):
    w = _wid()
    pltpu.sync_copy(src3d.at[w], sidx)
    pltpu.sync_copy(dst3d.at[w], didx)
    base = w * (CE // NW)

    pltpu.async_copy(p.at[sidx.at[0]], bp0, sp0)
    pltpu.async_copy(q.at[didx.at[0]], bq0, sq0)
    pltpu.async_copy(p.at[sidx.at[1]], bp1, sp1)
    pltpu.async_copy(q.at[didx.at[1]], bq1, sq1)

    def body(j2, carry):
        j = 2 * j2
        pltpu.make_async_copy(p.at[sidx.at[j]], bp0, sp0).wait()
        pltpu.sync_copy(bp0, pg.at[pl.ds(base + j * B, B)])
        pltpu.async_copy(p.at[sidx.at[j + 2]], bp0, sp0)
        pltpu.make_async_copy(q.at[didx.at[j]], bq0, sq0).wait()
        pltpu.sync_copy(bq0, qg.at[pl.ds(base + j * B, B)])
        pltpu.async_copy(q.at[didx.at[j + 2]], bq0, sq0)
        pltpu.make_async_copy(p.at[sidx.at[j + 1]], bp1, sp1).wait()
        pltpu.sync_copy(bp1, pg.at[pl.ds(base + (j + 1) * B, B)])
        pltpu.async_copy(p.at[sidx.at[j + 3]], bp1, sp1)
        pltpu.make_async_copy(q.at[didx.at[j + 1]], bq1, sq1).wait()
        pltpu.sync_copy(bq1, qg.at[pl.ds(base + (j + 1) * B, B)])
        pltpu.async_copy(q.at[didx.at[j + 3]], bq1, sq1)
        return carry

    lax.fori_loop(0, CRPT // 2 - 1, body, 0)
    j = CRPT - 2
    pltpu.make_async_copy(p.at[sidx.at[j]], bp0, sp0).wait()
    pltpu.sync_copy(bp0, pg.at[pl.ds(base + j * B, B)])
    pltpu.make_async_copy(q.at[didx.at[j]], bq0, sq0).wait()
    pltpu.sync_copy(bq0, qg.at[pl.ds(base + j * B, B)])
    pltpu.make_async_copy(p.at[sidx.at[j + 1]], bp1, sp1).wait()
    pltpu.sync_copy(bp1, pg.at[pl.ds(base + (j + 1) * B, B)])
    pltpu.make_async_copy(q.at[didx.at[j + 1]], bq1, sq1).wait()
    pltpu.sync_copy(bq1, qg.at[pl.ds(base + (j + 1) * B, B)])


_R = 1280


def _prep_body(x_ref, w_ref, d0_ref, d1_ref, y_ref, dis_ref):
    dis = lax.rsqrt(d0_ref[...] + d1_ref[...] + 1.0)
    y_ref[...] = jnp.dot(x_ref[...], w_ref[...],
                         preferred_element_type=_f32) * dis
    dis_ref[...] = dis


def _tc_prep(x_pad, W1, d0, d1):
    return pl.pallas_call(
        _prep_body,
        grid=(NPAD // _R,),
        in_specs=[
            pl.BlockSpec((_R, D), lambda i: (i, 0)),
            pl.BlockSpec((D, D), lambda i: (0, 0)),
            pl.BlockSpec((_R, 1), lambda i: (i, 0)),
            pl.BlockSpec((_R, 1), lambda i: (i, 0)),
        ],
        out_specs=[
            pl.BlockSpec((_R, D), lambda i: (i, 0)),
            pl.BlockSpec((_R, 1), lambda i: (i, 0)),
        ],
        out_shape=[
            jax.ShapeDtypeStruct((NPAD, D), _f32),
            jax.ShapeDtypeStruct((NPAD, 1), _f32),
        ],
    )(x_pad, W1, d0, d1)


def _layer_body(a0_ref, a1_ref, y_ref, dis_ref, b_ref, w_ref, o_ref):
    h = jnp.maximum(
        dis_ref[...] * (a0_ref[0] + a1_ref[0] + y_ref[...]) + b_ref[...], 0.0)
    o_ref[...] = jnp.dot(h, w_ref[...], preferred_element_type=_f32) * dis_ref[...]


def _tc_layer(agg, y, dis, b, W):
    return pl.pallas_call(
        _layer_body,
        grid=(NPAD // _R,),
        in_specs=[
            pl.BlockSpec((1, _R, D), lambda i: (0, i, 0)),
            pl.BlockSpec((1, _R, D), lambda i: (1, i, 0)),
            pl.BlockSpec((_R, D), lambda i: (i, 0)),
            pl.BlockSpec((_R, 1), lambda i: (i, 0)),
            pl.BlockSpec((1, D), lambda i: (0, 0)),
            pl.BlockSpec((D, D), lambda i: (0, 0)),
        ],
        out_specs=pl.BlockSpec((_R, D), lambda i: (i, 0)),
        out_shape=jax.ShapeDtypeStruct((NPAD, D), _f32),
    )(agg, agg, y, dis, b, W)


def _pq_body(a0_ref, a1_ref, y_ref, dis_ref, b_ref, wa_ref, wb_ref,
             p_ref, q_ref):
    h = jnp.maximum(
        dis_ref[...] * (a0_ref[0] + a1_ref[0] + y_ref[...]) + b_ref[...], 0.0)
    p_ref[...] = jnp.dot(h, wa_ref[...], preferred_element_type=_f32)
    q_ref[...] = jnp.dot(h, wb_ref[...], preferred_element_type=_f32)


def _tc_pq(agg, y, dis, b, WA, WB):
    return pl.pallas_call(
        _pq_body,
        grid=(NPAD // _R,),
        in_specs=[
            pl.BlockSpec((1, _R, D), lambda i: (0, i, 0)),
            pl.BlockSpec((1, _R, D), lambda i: (1, i, 0)),
            pl.BlockSpec((_R, D), lambda i: (i, 0)),
            pl.BlockSpec((_R, 1), lambda i: (i, 0)),
            pl.BlockSpec((1, D), lambda i: (0, 0)),
            pl.BlockSpec((D, D), lambda i: (0, 0)),
            pl.BlockSpec((D, D), lambda i: (0, 0)),
        ],
        out_specs=[
            pl.BlockSpec((_R, D), lambda i: (i, 0)),
            pl.BlockSpec((_R, D), lambda i: (i, 0)),
        ],
        out_shape=[
            jax.ShapeDtypeStruct((NPAD, D), _f32),
            jax.ShapeDtypeStruct((NPAD, D), _f32),
        ],
    )(agg, agg, y, dis, b, WA, WB)


_EB = 8000


def _edge_body(pg_ref, qg_ref, ef_ref, wc_ref, b1_ref, w2_ref, b2_ref,
               w3_ref, b3_ref, o_ref):
    z = pg_ref[...] + qg_ref[...] + jnp.dot(
        ef_ref[...], wc_ref[...], preferred_element_type=_f32) + b1_ref[...]
    z = jnp.maximum(z, 0.0)
    z = jnp.maximum(
        jnp.dot(z, w2_ref[...], preferred_element_type=_f32) + b2_ref[...], 0.0)
    o_ref[...] = jnp.dot(z, w3_ref[...], preferred_element_type=_f32) + b3_ref[...]


def _tc_edge(pg, qg, ef, WC, bm1, Wm2, bm2, Wm3, bm3):
    return pl.pallas_call(
        _edge_body,
        grid=(E // _EB,),
        in_specs=[
            pl.BlockSpec((_EB, D), lambda i: (i, 0)),
            pl.BlockSpec((_EB, D), lambda i: (i, 0)),
            pl.BlockSpec((_EB, 16), lambda i: (i, 0)),
            pl.BlockSpec((16, D), lambda i: (0, 0)),
            pl.BlockSpec((1, D), lambda i: (0, 0)),
            pl.BlockSpec((D, 64), lambda i: (0, 0)),
            pl.BlockSpec((1, 64), lambda i: (0, 0)),
            pl.BlockSpec((64, 1), lambda i: (0, 0)),
            pl.BlockSpec((1, 1), lambda i: (0, 0)),
        ],
        out_specs=pl.BlockSpec((_EB, 1), lambda i: (i, 0)),
        out_shape=jax.ShapeDtypeStruct((E, 1), _f32),
    )(pg, qg, ef, WC, bm1, Wm2, bm2, Wm3, bm3)


def kernel(x, edge_index, edge_feat, W1, b1, W2, b2, Wm1, bm1, Wm2, bm2,
           Wm3, bm3):
    npad_e = EPAD - E
    pad_iota = jnp.arange(npad_e, dtype=jnp.int32)
    src_pad = jnp.concatenate([edge_index[0], pad_iota % N])
    dst_pad = jnp.concatenate([edge_index[1], N + pad_iota % (NPAD - N)])
    src3d = src_pad.reshape(NW, RPT, B)
    dst3d = dst_pad.reshape(NW, RPT, B)
    x_pad = jnp.pad(x, ((0, NPAD - N), (0, 0)))
    zeros1 = jnp.zeros((STRIP,), _f32)

    d0, d1 = _sc_degree(dst3d, zeros1)
    y1, dis = _tc_prep(x_pad, W1, d0.reshape(NPAD, 1), d1.reshape(NPAD, 1))
    agg1 = _sc_agg(y1, src3d, dst3d)
    y2 = _tc_layer(agg1, y1, dis, b1.reshape(1, D), W2)
    agg2 = _sc_agg(y2, src3d, dst3d)
    p, q = _tc_pq(agg2, y2, dis, b2.reshape(1, D), Wm1[:D], Wm1[D:2 * D])
    pg, qg = _sc_gather2(p, q, src3d, dst3d)
    return _tc_edge(pg, qg, edge_feat, Wm1[2 * D:], bm1.reshape(1, D),
                    Wm2, bm2.reshape(1, 64), Wm3, bm3.reshape(1, 1))


# speedup vs baseline: 1.0640x; 1.0023x over previous
import functools

import jax
import jax.numpy as jnp
from jax import lax
from jax.experimental import pallas as pl
from jax.experimental.pallas import tpu as pltpu
from jax.experimental.pallas import tpu_sc as plsc

N = 10000
E = 320000
D = 128
NPAD = 10240
STRIP = NPAD // 16
NC, NS = 2, 16
NW = NC * NS
B = 128
EPAD = NW * 80 * B
EPT = EPAD // NW
RPT = EPT // B
KCH = 1
CE = EPAD // KCH
CRPT = RPT // KCH

_MESH = plsc.VectorSubcoreMesh(core_axis_name="c", subcore_axis_name="s")
_f32 = jnp.float32


def _wid():
    return lax.axis_index("s") * NC + lax.axis_index("c")


@functools.partial(
    pl.kernel,
    out_type=(jax.ShapeDtypeStruct((NPAD,), _f32),
              jax.ShapeDtypeStruct((NPAD,), _f32)),
    mesh=_MESH,
    scratch_types=[
        pltpu.VMEM((RPT, B), jnp.int32),
        pltpu.VMEM((B,), _f32),
        pltpu.VMEM_SHARED((NPAD,), _f32),
    ],
)
def _sc_degree(dst3d, zeros1, out0, out1, didx, ones_v, acc):
    c = lax.axis_index("c")
    s = lax.axis_index("s")
    pltpu.sync_copy(zeros1, acc.at[pl.ds(s * STRIP, STRIP)])
    pltpu.sync_copy(dst3d.at[_wid()], didx)
    for k in range(B // 16):
        ones_v[pl.ds(k * 16, 16)] = jnp.ones((16,), _f32)
    plsc.subcore_barrier()

    def body(j, carry):
        pltpu.sync_copy(ones_v, acc.at[didx.at[j]], add=True)
        return carry

    lax.fori_loop(0, RPT, body, 0)
    plsc.subcore_barrier()

    @pl.when(c == 0)
    def _():
        pltpu.sync_copy(acc.at[pl.ds(s * STRIP, STRIP)],
                        out0.at[pl.ds(s * STRIP, STRIP)])

    @pl.when(c == 1)
    def _():
        pltpu.sync_copy(acc.at[pl.ds(s * STRIP, STRIP)],
                        out1.at[pl.ds(s * STRIP, STRIP)])


@functools.partial(
    pl.kernel,
    out_type=jax.ShapeDtypeStruct((NC, NPAD, D), _f32),
    mesh=_MESH,
    scratch_types=[
        pltpu.VMEM((RPT, B), jnp.int32),
        pltpu.VMEM((RPT, B), jnp.int32),
        pltpu.VMEM((B, D), _f32),
        pltpu.VMEM_SHARED((NPAD, D), _f32),
        pltpu.SemaphoreType.DMA,
    ],
)
def _sc_agg(y, src3d, dst3d, out, sidx, didx, rows, acc, sem):
    c = lax.axis_index("c")
    s = lax.axis_index("s")
    w = _wid()
    for i in range(B):
        for k in range(D // 16):
            rows[i, pl.ds(k * 16, 16)] = jnp.zeros((16,), _f32)
    for k in range(STRIP // B):
        pltpu.sync_copy(rows, acc.at[pl.ds(s * STRIP + k * B, B)])
    pltpu.sync_copy(src3d.at[w], sidx)
    pltpu.sync_copy(dst3d.at[w], didx)
    plsc.subcore_barrier()

    def body(j, carry):
        pltpu.async_copy(y.at[sidx.at[j]], rows, sem).wait()
        pltpu.sync_copy(rows, acc.at[didx.at[j]], add=True)
        return carry

    lax.fori_loop(0, RPT, body, 0)
    plsc.subcore_barrier()
    pltpu.sync_copy(acc.at[pl.ds(s * STRIP, STRIP)],
                    out.at[c, pl.ds(s * STRIP, STRIP)])


@functools.partial(
    pl.kernel,
    out_type=(jax.ShapeDtypeStruct((CE, D), _f32),
              jax.ShapeDtypeStruct((CE, D), _f32)),
    mesh=_MESH,
    scratch_types=[
        pltpu.VMEM((CRPT, B), jnp.int32),
        pltpu.VMEM((CRPT, B), jnp.int32),
        pltpu.VMEM((B, D), _f32),
        pltpu.VMEM((B, D), _f32),
        pltpu.VMEM((B, D), _f32),
        pltpu.VMEM((B, D), _f32),
        pltpu.SemaphoreType.DMA,
        pltpu.SemaphoreType.DMA,
        pltpu.SemaphoreType.DMA,
        pltpu.SemaphoreType.DMA,
    ],
)
def _sc_gather2(p, q, src3d, dst3d, pg, qg,
                sidx, didx, bp0, bp1, bq0, bq1, sp0, sp1, sq0, sq1):
    w = _wid()
    pltpu.sync_copy(src3d.at[w], sidx)
    pltpu.sync_copy(dst3d.at[w], didx)
    base = w * (CE // NW)

    pltpu.async_copy(p.at[sidx.at[0]], bp0, sp0)
    pltpu.async_copy(q.at[didx.at[0]], bq0, sq0)
    pltpu.async_copy(p.at[sidx.at[1]], bp1, sp1)
    pltpu.async_copy(q.at[didx.at[1]], bq1, sq1)

    def body(j2, carry):
        j = 2 * j2
        pltpu.make_async_copy(p.at[sidx.at[j]], bp0, sp0).wait()
        pltpu.sync_copy(bp0, pg.at[pl.ds(base + j * B, B)])
        pltpu.async_copy(p.at[sidx.at[j + 2]], bp0, sp0)
        pltpu.make_async_copy(q.at[didx.at[j]], bq0, sq0).wait()
        pltpu.sync_copy(bq0, qg.at[pl.ds(base + j * B, B)])
        pltpu.async_copy(q.at[didx.at[j + 2]], bq0, sq0)
        pltpu.make_async_copy(p.at[sidx.at[j + 1]], bp1, sp1).wait()
        pltpu.sync_copy(bp1, pg.at[pl.ds(base + (j + 1) * B, B)])
        pltpu.async_copy(p.at[sidx.at[j + 3]], bp1, sp1)
        pltpu.make_async_copy(q.at[didx.at[j + 1]], bq1, sq1).wait()
        pltpu.sync_copy(bq1, qg.at[pl.ds(base + (j + 1) * B, B)])
        pltpu.async_copy(q.at[didx.at[j + 3]], bq1, sq1)
        return carry

    lax.fori_loop(0, CRPT // 2 - 1, body, 0)
    j = CRPT - 2
    pltpu.make_async_copy(p.at[sidx.at[j]], bp0, sp0).wait()
    pltpu.sync_copy(bp0, pg.at[pl.ds(base + j * B, B)])
    pltpu.make_async_copy(q.at[didx.at[j]], bq0, sq0).wait()
    pltpu.sync_copy(bq0, qg.at[pl.ds(base + j * B, B)])
    pltpu.make_async_copy(p.at[sidx.at[j + 1]], bp1, sp1).wait()
    pltpu.sync_copy(bp1, pg.at[pl.ds(base + (j + 1) * B, B)])
    pltpu.make_async_copy(q.at[didx.at[j + 1]], bq1, sq1).wait()
    pltpu.sync_copy(bq1, qg.at[pl.ds(base + (j + 1) * B, B)])


_R = 2048


def _prep_body(x_ref, w_ref, d0_ref, d1_ref, y_ref, dis_ref):
    dis = lax.rsqrt(d0_ref[...] + d1_ref[...] + 1.0)
    y_ref[...] = jnp.dot(x_ref[...], w_ref[...],
                         preferred_element_type=_f32) * dis
    dis_ref[...] = dis


def _tc_prep(x_pad, W1, d0, d1):
    return pl.pallas_call(
        _prep_body,
        grid=(NPAD // _R,),
        in_specs=[
            pl.BlockSpec((_R, D), lambda i: (i, 0)),
            pl.BlockSpec((D, D), lambda i: (0, 0)),
            pl.BlockSpec((_R, 1), lambda i: (i, 0)),
            pl.BlockSpec((_R, 1), lambda i: (i, 0)),
        ],
        out_specs=[
            pl.BlockSpec((_R, D), lambda i: (i, 0)),
            pl.BlockSpec((_R, 1), lambda i: (i, 0)),
        ],
        out_shape=[
            jax.ShapeDtypeStruct((NPAD, D), _f32),
            jax.ShapeDtypeStruct((NPAD, 1), _f32),
        ],
    )(x_pad, W1, d0, d1)


def _layer_body(a0_ref, a1_ref, y_ref, dis_ref, b_ref, w_ref, o_ref):
    h = jnp.maximum(
        dis_ref[...] * (a0_ref[0] + a1_ref[0] + y_ref[...]) + b_ref[...], 0.0)
    o_ref[...] = jnp.dot(h, w_ref[...], preferred_element_type=_f32) * dis_ref[...]


def _tc_layer(agg, y, dis, b, W):
    return pl.pallas_call(
        _layer_body,
        grid=(NPAD // _R,),
        in_specs=[
            pl.BlockSpec((1, _R, D), lambda i: (0, i, 0)),
            pl.BlockSpec((1, _R, D), lambda i: (1, i, 0)),
            pl.BlockSpec((_R, D), lambda i: (i, 0)),
            pl.BlockSpec((_R, 1), lambda i: (i, 0)),
            pl.BlockSpec((1, D), lambda i: (0, 0)),
            pl.BlockSpec((D, D), lambda i: (0, 0)),
        ],
        out_specs=pl.BlockSpec((_R, D), lambda i: (i, 0)),
        out_shape=jax.ShapeDtypeStruct((NPAD, D), _f32),
    )(agg, agg, y, dis, b, W)


def _pq_body(a0_ref, a1_ref, y_ref, dis_ref, b_ref, wa_ref, wb_ref,
             p_ref, q_ref):
    h = jnp.maximum(
        dis_ref[...] * (a0_ref[0] + a1_ref[0] + y_ref[...]) + b_ref[...], 0.0)
    p_ref[...] = jnp.dot(h, wa_ref[...], preferred_element_type=_f32)
    q_ref[...] = jnp.dot(h, wb_ref[...], preferred_element_type=_f32)


def _tc_pq(agg, y, dis, b, WA, WB):
    return pl.pallas_call(
        _pq_body,
        grid=(NPAD // _R,),
        in_specs=[
            pl.BlockSpec((1, _R, D), lambda i: (0, i, 0)),
            pl.BlockSpec((1, _R, D), lambda i: (1, i, 0)),
            pl.BlockSpec((_R, D), lambda i: (i, 0)),
            pl.BlockSpec((_R, 1), lambda i: (i, 0)),
            pl.BlockSpec((1, D), lambda i: (0, 0)),
            pl.BlockSpec((D, D), lambda i: (0, 0)),
            pl.BlockSpec((D, D), lambda i: (0, 0)),
        ],
        out_specs=[
            pl.BlockSpec((_R, D), lambda i: (i, 0)),
            pl.BlockSpec((_R, D), lambda i: (i, 0)),
        ],
        out_shape=[
            jax.ShapeDtypeStruct((NPAD, D), _f32),
            jax.ShapeDtypeStruct((NPAD, D), _f32),
        ],
    )(agg, agg, y, dis, b, WA, WB)


_EB = 10000


def _edge_body(pg_ref, qg_ref, ef_ref, wc_ref, b1_ref, w2_ref, b2_ref,
               w3_ref, b3_ref, o_ref):
    z = pg_ref[...] + qg_ref[...] + jnp.dot(
        ef_ref[...], wc_ref[...], preferred_element_type=_f32) + b1_ref[...]
    z = jnp.maximum(z, 0.0)
    z = jnp.maximum(
        jnp.dot(z, w2_ref[...], preferred_element_type=_f32) + b2_ref[...], 0.0)
    o_ref[...] = jnp.dot(z, w3_ref[...], preferred_element_type=_f32) + b3_ref[...]


def _tc_edge(pg, qg, ef, WC, bm1, Wm2, bm2, Wm3, bm3):
    return pl.pallas_call(
        _edge_body,
        grid=(E // _EB,),
        in_specs=[
            pl.BlockSpec((_EB, D), lambda i: (i, 0)),
            pl.BlockSpec((_EB, D), lambda i: (i, 0)),
            pl.BlockSpec((_EB, 16), lambda i: (i, 0)),
            pl.BlockSpec((16, D), lambda i: (0, 0)),
            pl.BlockSpec((1, D), lambda i: (0, 0)),
            pl.BlockSpec((D, 64), lambda i: (0, 0)),
            pl.BlockSpec((1, 64), lambda i: (0, 0)),
            pl.BlockSpec((64, 1), lambda i: (0, 0)),
            pl.BlockSpec((1, 1), lambda i: (0, 0)),
        ],
        out_specs=pl.BlockSpec((_EB, 1), lambda i: (i, 0)),
        out_shape=jax.ShapeDtypeStruct((E, 1), _f32),
    )(pg, qg, ef, WC, bm1, Wm2, bm2, Wm3, bm3)


def kernel(x, edge_index, edge_feat, W1, b1, W2, b2, Wm1, bm1, Wm2, bm2,
           Wm3, bm3):
    npad_e = EPAD - E
    pad_iota = jnp.arange(npad_e, dtype=jnp.int32)
    src_pad = jnp.concatenate([edge_index[0], pad_iota % N])
    dst_pad = jnp.concatenate([edge_index[1], N + pad_iota % (NPAD - N)])
    src3d = src_pad.reshape(NW, RPT, B)
    dst3d = dst_pad.reshape(NW, RPT, B)
    x_pad = jnp.pad(x, ((0, NPAD - N), (0, 0)))
    zeros1 = jnp.zeros((STRIP,), _f32)

    d0, d1 = _sc_degree(dst3d, zeros1)
    y1, dis = _tc_prep(x_pad, W1, d0.reshape(NPAD, 1), d1.reshape(NPAD, 1))
    agg1 = _sc_agg(y1, src3d, dst3d)
    y2 = _tc_layer(agg1, y1, dis, b1.reshape(1, D), W2)
    agg2 = _sc_agg(y2, src3d, dst3d)
    p, q = _tc_pq(agg2, y2, dis, b2.reshape(1, D), Wm1[:D], Wm1[D:2 * D])
    pg, qg = _sc_gather2(p, q, src3d, dst3d)
    return _tc_edge(pg, qg, edge_feat, Wm1[2 * D:], bm1.reshape(1, D),
                    Wm2, bm2.reshape(1, 64), Wm3, bm3.reshape(1, 1))
